# baseline (speedup 1.0000x reference)
import functools

import jax
import jax.numpy as jnp
from jax import lax
from jax.experimental import pallas as pl
from jax.experimental.pallas import tpu as pltpu
from jax.experimental.pallas import tpu_sc as plsc

D = 64
DP = 128
T = 200
LANES = 16
IDX_MINOR = 128
K_STREAMS = 2
CHUNK = K_STREAMS * IDX_MINOR
STRIPE = 128


def _sc_transpose(table_t, tail128):
    nf, vocab = table_t.shape
    assert nf == D
    info = plsc.get_sparse_core_info()
    nw = info.num_cores * info.num_subcores
    nc = info.num_cores
    n_full = vocab // STRIPE
    tail = vocab - n_full * STRIPE
    n_out = (n_full + (1 if tail else 0)) * STRIPE
    mesh = plsc.VectorSubcoreMesh(core_axis_name="c", subcore_axis_name="s")

    @functools.partial(
        pl.kernel,
        mesh=mesh,
        compiler_params=pltpu.CompilerParams(needs_layout_passes=False),
        out_type=jax.ShapeDtypeStruct((n_out, DP), jnp.float32),
        scratch_types=[
            pltpu.VMEM((D, STRIPE), jnp.float32),
            pltpu.VMEM((D, STRIPE), jnp.float32),
            pltpu.VMEM((STRIPE, DP), jnp.float32),
            pltpu.VMEM((STRIPE, DP), jnp.float32),
            pltpu.SemaphoreType.DMA,
            pltpu.SemaphoreType.DMA,
            pltpu.SemaphoreType.DMA,
            pltpu.SemaphoreType.DMA,
        ],
    )
    def k(tt_hbm, tail_hbm, out_hbm, in0, in1, st0, st1, si0, si1, so0, so1):
        wid = lax.axis_index("s") * nc + lax.axis_index("c")
        in_v = (in0, in1)
        st_v = (st0, st1)
        si = (si0, si1)
        so = (so0, so1)
        my_stripes = (n_full - wid + nw - 1) // nw

        def in_start(s, p, w=STRIPE):
            pltpu.async_copy(
                tt_hbm.at[:, pl.ds(s * STRIPE, w)],
                in_v[p].at[:, pl.ds(0, w)],
                si[p],
            )

        def in_wait(s, p, w=STRIPE):
            pltpu.make_async_copy(
                tt_hbm.at[:, pl.ds(s * STRIPE, w)],
                in_v[p].at[:, pl.ds(0, w)],
                si[p],
            ).wait()

        def out_start(s, p, w=STRIPE):
            pltpu.async_copy(
                st_v[p].at[pl.ds(0, w)],
                out_hbm.at[pl.ds(s * STRIPE, w)],
                so[p],
            )

        def out_wait(s, p, w=STRIPE):
            pltpu.make_async_copy(
                st_v[p].at[pl.ds(0, w)],
                out_hbm.at[pl.ds(s * STRIPE, w)],
                so[p],
            ).wait()

        iotas = lax.iota(jnp.int32, LANES)

        def shuffle(p, w=STRIPE):
            for f in range(D):
                fcol = jnp.full((LANES,), f, jnp.int32)
                for b in range(w // LANES):
                    x = in_v[p][f, pl.ds(b * LANES, LANES)]
                    plsc.store_scatter(
                        st_v[p], [iotas + b * LANES, fcol], x)

        def body(i, _):
            for p in (0, 1):
                j = i * 2 + p
                s = wid + j * nw

                @pl.when(j < my_stripes)
                def _():
                    in_wait(s, p)

                    @pl.when(j >= 2)
                    def _():
                        out_wait(s - 2 * nw, p)

                    shuffle(p)

                    @pl.when(j + 2 < my_stripes)
                    def _():
                        in_start(s + 2 * nw, p)

                    out_start(s, p)

            return 0

        @pl.when(my_stripes > 0)
        def _():
            in_start(wid, 0)

        @pl.when(my_stripes > 1)
        def _():
            in_start(wid + nw, 1)

        max_iters = (n_full + nw - 1) // nw
        lax.fori_loop(0, (max_iters + 1) // 2, body, 0, unroll=False)

        lj = my_stripes - 1
        for p in (0, 1):
            lj_p = jnp.where(lj % 2 == p, lj, lj - 1)

            @pl.when(lj_p >= 0)
            def _():
                out_wait(wid + lj_p * nw, p)

        if tail:
            @pl.when(wid == 0)
            def _():
                pltpu.sync_copy(tail_hbm, in_v[0])
                shuffle(0)
                out_start(n_full, 0)
                out_wait(n_full, 0)

    return k(table_t, tail128)


def _sc_gather(tok_tmajor, table_p, pos_flat, batch):
    n_rows = tok_tmajor.shape[0]
    n_t = n_rows // batch
    bblocks = batch // CHUNK
    n_units = n_t * bblocks
    info = plsc.get_sparse_core_info()
    nc = info.num_cores
    nw = nc * info.num_subcores
    units_per_worker = n_units // nw
    assert n_units % nw == 0
    assert units_per_worker % 2 == 0

    mesh = plsc.VectorSubcoreMesh(core_axis_name="c", subcore_axis_name="s")

    @functools.partial(
        pl.kernel,
        mesh=mesh,
        compiler_params=pltpu.CompilerParams(needs_layout_passes=False),
        out_type=jax.ShapeDtypeStruct((n_t, D, batch), jnp.float32),
        scratch_types=[
            pltpu.VMEM((CHUNK,), jnp.int32),
            pltpu.VMEM((CHUNK,), jnp.int32),
            pltpu.VMEM((CHUNK, DP), jnp.float32),
            pltpu.VMEM((CHUNK, DP), jnp.float32),
            pltpu.VMEM((D, CHUNK), jnp.float32),
            pltpu.VMEM((D, CHUNK), jnp.float32),
            pltpu.VMEM((T * D,), jnp.float32),
            pltpu.SemaphoreType.DMA,
            pltpu.SemaphoreType.DMA,
            pltpu.SemaphoreType.DMA,
            pltpu.SemaphoreType.DMA,
            pltpu.SemaphoreType.DMA,
            pltpu.SemaphoreType.DMA,
        ],
    )
    def k(tok_hbm, table_hbm, pos_hbm, out_hbm, idx0, idx1, wide0, wide1,
          slab0, slab1, pos_v, sg0, sg1, si0, si1, so0, so1):
        wid = lax.axis_index("s") * nc + lax.axis_index("c")
        pltpu.sync_copy(pos_hbm, pos_v)
        ubase = wid * units_per_worker
        idx_v = (idx0, idx1)
        wide_v = (wide0, wide1)
        slab_v = (slab0, slab1)
        sg = (sg0, sg1)
        si = (si0, si1)
        so = (so0, so1)

        def idx_start(u, p):
            pltpu.async_copy(
                tok_hbm.at[pl.ds((ubase + u) * CHUNK, CHUNK)], idx_v[p], si[p])

        def idx_wait(u, p):
            pltpu.make_async_copy(
                tok_hbm.at[pl.ds((ubase + u) * CHUNK, CHUNK)], idx_v[p], si[p]
            ).wait()

        def gathers_start(p):
            for j in range(K_STREAMS):
                pltpu.async_copy(
                    table_hbm.at[idx_v[p].at[pl.ds(j * IDX_MINOR, IDX_MINOR)]],
                    wide_v[p].at[pl.ds(j * IDX_MINOR, IDX_MINOR)],
                    sg[p],
                )

        def gathers_wait(p):
            for j in range(K_STREAMS):
                pltpu.make_async_copy(
                    table_hbm.at[idx_v[p].at[pl.ds(j * IDX_MINOR, IDX_MINOR)]],
                    wide_v[p].at[pl.ds(j * IDX_MINOR, IDX_MINOR)],
                    sg[p],
                ).wait()

        def unit_t(u):
            return (ubase + u) // bblocks

        def unit_b0(u):
            return ((ubase + u) % bblocks) * CHUNK

        def out_start(u, p):
            pltpu.async_copy(
                slab_v[p],
                out_hbm.at[unit_t(u), :, pl.ds(unit_b0(u), CHUNK)],
                so[p],
            )

        def out_wait(u, p):
            pltpu.make_async_copy(
                slab_v[p],
                out_hbm.at[unit_t(u), :, pl.ds(unit_b0(u), CHUNK)],
                so[p],
            ).wait()

        iotas = lax.iota(jnp.int32, LANES)

        def transpose_add(u, p):
            t = unit_t(u)
            toff = pl.multiple_of(t * D, D)
            pos_parts = [
                pos_v[pl.ds(toff + cc * LANES, LANES)]
                for cc in range(D // LANES)
            ]
            rows_idx = [iotas + cc * LANES for cc in range(D // LANES)]

            def row_body(b, _):
                bcol = jnp.full((LANES,), 0, jnp.int32) + b
                for cc in range(D // LANES):
                    x = wide_v[p][b, pl.ds(cc * LANES, LANES)]
                    plsc.store_scatter(
                        slab_v[p], [rows_idx[cc], bcol], x + pos_parts[cc])
                return 0

            lax.fori_loop(0, CHUNK, row_body, 0, unroll=8)

        idx_start(0, 0)
        idx_start(1, 1)
        idx_wait(0, 0)
        gathers_start(0)

        def pair_body(i, _):
            for p in (0, 1):
                u = i * 2 + p

                @pl.when(u + 1 < units_per_worker)
                def _():
                    idx_wait(u + 1, 1 - p)
                    gathers_start(1 - p)

                gathers_wait(p)

                @pl.when(u >= 2)
                def _():
                    out_wait(u - 2, p)

                transpose_add(u, p)
                out_start(u, p)

                @pl.when(u + 2 < units_per_worker)
                def _():
                    idx_start(u + 2, p)

            return 0

        lax.fori_loop(0, units_per_worker // 2, pair_body, 0)
        out_wait(units_per_worker - 2, 0)
        out_wait(units_per_worker - 1, 1)

    return k(tok_tmajor, table_p, pos_flat)


def kernel(tokens, token_embedding, position_embedding):
    b, t = tokens.shape
    tok_tmajor = tokens.T.astype(jnp.int32).reshape(-1)
    pos_flat = position_embedding.reshape(-1)
    table_t = token_embedding.T
    vocab = table_t.shape[1]
    n_full = vocab // STRIPE
    tail = vocab - n_full * STRIPE
    tail128 = jnp.pad(
        table_t[:, n_full * STRIPE:], ((0, 0), (0, STRIPE - tail)))
    table_p = _sc_transpose(table_t, tail128)
    out = _sc_gather(tok_tmajor, table_p, pos_flat, b)
    return jnp.transpose(out, (2, 0, 1))


# speedup vs baseline: 2.1495x; 2.1495x over previous
import functools

import jax
import jax.numpy as jnp
from jax import lax
from jax.experimental import pallas as pl
from jax.experimental.pallas import tpu as pltpu
from jax.experimental.pallas import tpu_sc as plsc

D = 64
DP = 128
T = 200
LANES = 16
IDX_MINOR = 128
K_STREAMS = 2
CHUNK = K_STREAMS * IDX_MINOR
STRIPE = 128


def _sc_transpose(table_t, tail128):
    nf, vocab = table_t.shape
    assert nf == D
    info = plsc.get_sparse_core_info()
    nw = info.num_cores * info.num_subcores
    nc = info.num_cores
    n_full = vocab // STRIPE
    tail = vocab - n_full * STRIPE
    n_out = (n_full + (1 if tail else 0)) * STRIPE
    mesh = plsc.VectorSubcoreMesh(core_axis_name="c", subcore_axis_name="s")

    @functools.partial(
        pl.kernel,
        mesh=mesh,
        compiler_params=pltpu.CompilerParams(needs_layout_passes=False),
        out_type=jax.ShapeDtypeStruct((n_out, DP), jnp.float32),
        scratch_types=[
            pltpu.VMEM((D, STRIPE), jnp.float32),
            pltpu.VMEM((D, STRIPE), jnp.float32),
            pltpu.VMEM((STRIPE, DP), jnp.float32),
            pltpu.VMEM((STRIPE, DP), jnp.float32),
            pltpu.SemaphoreType.DMA,
            pltpu.SemaphoreType.DMA,
            pltpu.SemaphoreType.DMA,
            pltpu.SemaphoreType.DMA,
        ],
    )
    def k(tt_hbm, tail_hbm, out_hbm, in0, in1, st0, st1, si0, si1, so0, so1):
        wid = lax.axis_index("s") * nc + lax.axis_index("c")
        in_v = (in0, in1)
        st_v = (st0, st1)
        si = (si0, si1)
        so = (so0, so1)
        my_stripes = (n_full - wid + nw - 1) // nw

        def in_start(s, p, w=STRIPE):
            pltpu.async_copy(
                tt_hbm.at[:, pl.ds(s * STRIPE, w)],
                in_v[p].at[:, pl.ds(0, w)],
                si[p],
            )

        def in_wait(s, p, w=STRIPE):
            pltpu.make_async_copy(
                tt_hbm.at[:, pl.ds(s * STRIPE, w)],
                in_v[p].at[:, pl.ds(0, w)],
                si[p],
            ).wait()

        def out_start(s, p, w=STRIPE):
            pltpu.async_copy(
                st_v[p].at[pl.ds(0, w)],
                out_hbm.at[pl.ds(s * STRIPE, w)],
                so[p],
            )

        def out_wait(s, p, w=STRIPE):
            pltpu.make_async_copy(
                st_v[p].at[pl.ds(0, w)],
                out_hbm.at[pl.ds(s * STRIPE, w)],
                so[p],
            ).wait()

        iotas = lax.iota(jnp.int32, LANES)

        def shuffle(p):
            def vb_body(vb, _):
                vvec = vb * LANES + iotas

                def d_body(d, _):
                    rot = (d + iotas) % LANES
                    for fb in range(D // LANES):
                        fvec = fb * LANES + rot
                        x = plsc.load_gather(in_v[p], [fvec, vvec])
                        plsc.store_scatter(st_v[p], [vvec, fvec], x)
                    return 0

                return lax.fori_loop(0, LANES, d_body, 0, unroll=2)

            lax.fori_loop(0, STRIPE // LANES, vb_body, 0, unroll=False)

        def body(i, _):
            for p in (0, 1):
                j = i * 2 + p
                s = wid + j * nw

                @pl.when(j < my_stripes)
                def _():
                    in_wait(s, p)

                    @pl.when(j >= 2)
                    def _():
                        out_wait(s - 2 * nw, p)

                    shuffle(p)

                    @pl.when(j + 2 < my_stripes)
                    def _():
                        in_start(s + 2 * nw, p)

                    out_start(s, p)

            return 0

        @pl.when(my_stripes > 0)
        def _():
            in_start(wid, 0)

        @pl.when(my_stripes > 1)
        def _():
            in_start(wid + nw, 1)

        max_iters = (n_full + nw - 1) // nw
        lax.fori_loop(0, (max_iters + 1) // 2, body, 0, unroll=False)

        lj = my_stripes - 1
        for p in (0, 1):
            lj_p = jnp.where(lj % 2 == p, lj, lj - 1)

            @pl.when(lj_p >= 0)
            def _():
                out_wait(wid + lj_p * nw, p)

        if tail:
            @pl.when(wid == 0)
            def _():
                pltpu.sync_copy(tail_hbm, in_v[0])
                shuffle(0)
                out_start(n_full, 0)
                out_wait(n_full, 0)

    return k(table_t, tail128)


def _sc_gather(tok_tmajor, table_p, pos_flat, batch):
    n_rows = tok_tmajor.shape[0]
    n_t = n_rows // batch
    bblocks = batch // CHUNK
    n_units = n_t * bblocks
    info = plsc.get_sparse_core_info()
    nc = info.num_cores
    nw = nc * info.num_subcores
    units_per_worker = n_units // nw
    assert n_units % nw == 0
    assert units_per_worker % 2 == 0

    mesh = plsc.VectorSubcoreMesh(core_axis_name="c", subcore_axis_name="s")

    @functools.partial(
        pl.kernel,
        mesh=mesh,
        compiler_params=pltpu.CompilerParams(needs_layout_passes=False),
        out_type=jax.ShapeDtypeStruct((n_t, D, batch), jnp.float32),
        scratch_types=[
            pltpu.VMEM((CHUNK,), jnp.int32),
            pltpu.VMEM((CHUNK,), jnp.int32),
            pltpu.VMEM((CHUNK, DP), jnp.float32),
            pltpu.VMEM((CHUNK, DP), jnp.float32),
            pltpu.VMEM((D, CHUNK), jnp.float32),
            pltpu.VMEM((D, CHUNK), jnp.float32),
            pltpu.VMEM((T * D,), jnp.float32),
            pltpu.SemaphoreType.DMA,
            pltpu.SemaphoreType.DMA,
            pltpu.SemaphoreType.DMA,
            pltpu.SemaphoreType.DMA,
            pltpu.SemaphoreType.DMA,
            pltpu.SemaphoreType.DMA,
        ],
    )
    def k(tok_hbm, table_hbm, pos_hbm, out_hbm, idx0, idx1, wide0, wide1,
          slab0, slab1, pos_v, sg0, sg1, si0, si1, so0, so1):
        wid = lax.axis_index("s") * nc + lax.axis_index("c")
        pltpu.sync_copy(pos_hbm, pos_v)
        ubase = wid * units_per_worker
        idx_v = (idx0, idx1)
        wide_v = (wide0, wide1)
        slab_v = (slab0, slab1)
        sg = (sg0, sg1)
        si = (si0, si1)
        so = (so0, so1)

        def idx_start(u, p):
            pltpu.async_copy(
                tok_hbm.at[pl.ds((ubase + u) * CHUNK, CHUNK)], idx_v[p], si[p])

        def idx_wait(u, p):
            pltpu.make_async_copy(
                tok_hbm.at[pl.ds((ubase + u) * CHUNK, CHUNK)], idx_v[p], si[p]
            ).wait()

        def gathers_start(p):
            for j in range(K_STREAMS):
                pltpu.async_copy(
                    table_hbm.at[idx_v[p].at[pl.ds(j * IDX_MINOR, IDX_MINOR)]],
                    wide_v[p].at[pl.ds(j * IDX_MINOR, IDX_MINOR)],
                    sg[p],
                )

        def gathers_wait(p):
            for j in range(K_STREAMS):
                pltpu.make_async_copy(
                    table_hbm.at[idx_v[p].at[pl.ds(j * IDX_MINOR, IDX_MINOR)]],
                    wide_v[p].at[pl.ds(j * IDX_MINOR, IDX_MINOR)],
                    sg[p],
                ).wait()

        def unit_t(u):
            return (ubase + u) // bblocks

        def unit_b0(u):
            return ((ubase + u) % bblocks) * CHUNK

        def out_start(u, p):
            pltpu.async_copy(
                slab_v[p],
                out_hbm.at[unit_t(u), :, pl.ds(unit_b0(u), CHUNK)],
                so[p],
            )

        def out_wait(u, p):
            pltpu.make_async_copy(
                slab_v[p],
                out_hbm.at[unit_t(u), :, pl.ds(unit_b0(u), CHUNK)],
                so[p],
            ).wait()

        iotas = lax.iota(jnp.int32, LANES)

        def transpose_add(u, p):
            t = unit_t(u)
            toff = pl.multiple_of(t * D, D)

            def d_body(d, _):
                rot = (d + iotas) % LANES
                for k in range(D // LANES):
                    fvec = k * LANES + rot
                    posv = plsc.load_gather(pos_v, [toff + fvec])

                    def bb_body(bb, _):
                        bvec = bb * LANES + iotas
                        x = plsc.load_gather(wide_v[p], [bvec, fvec])
                        plsc.store_scatter(
                            slab_v[p], [fvec, bvec], x + posv)
                        return 0

                    lax.fori_loop(0, CHUNK // LANES, bb_body, 0, unroll=4)
                return 0

            lax.fori_loop(0, LANES, d_body, 0, unroll=False)

        idx_start(0, 0)
        idx_start(1, 1)
        idx_wait(0, 0)
        gathers_start(0)

        def pair_body(i, _):
            for p in (0, 1):
                u = i * 2 + p

                @pl.when(u + 1 < units_per_worker)
                def _():
                    idx_wait(u + 1, 1 - p)
                    gathers_start(1 - p)

                gathers_wait(p)

                @pl.when(u >= 2)
                def _():
                    out_wait(u - 2, p)

                transpose_add(u, p)
                out_start(u, p)

                @pl.when(u + 2 < units_per_worker)
                def _():
                    idx_start(u + 2, p)

            return 0

        lax.fori_loop(0, units_per_worker // 2, pair_body, 0)
        out_wait(units_per_worker - 2, 0)
        out_wait(units_per_worker - 1, 1)

    return k(tok_tmajor, table_p, pos_flat)


def kernel(tokens, token_embedding, position_embedding):
    b, t = tokens.shape
    tok_tmajor = tokens.T.astype(jnp.int32).reshape(-1)
    pos_flat = position_embedding.reshape(-1)
    table_t = token_embedding.T
    vocab = table_t.shape[1]
    n_full = vocab // STRIPE
    tail = vocab - n_full * STRIPE
    tail128 = jnp.pad(
        table_t[:, n_full * STRIPE:], ((0, 0), (0, STRIPE - tail)))
    table_p = _sc_transpose(table_t, tail128)
    out = _sc_gather(tok_tmajor, table_p, pos_flat, b)
    return jnp.transpose(out, (2, 0, 1))
